# Initial kernel scaffold; baseline (speedup 1.0000x reference)
import jax
import jax.numpy as jnp
from jax.experimental import pallas as pl


def kernel(x, y, relative_pos):
    raise NotImplementedError("write your pallas kernel here")


# speedup vs baseline: 24.3642x; 24.3642x over previous
import jax
import jax.numpy as jnp
from jax.experimental import pallas as pl

_K = 9
_BLOCK_R = 256
_OUT_W = 16


def _knn_block(xt_ref, y_ref, rel_ref, out_ref):
    xb = xt_ref[...]
    xn = xb / jnp.maximum(
        jnp.sqrt(jnp.sum(xb * xb, axis=1, keepdims=True)), 1e-12)
    yb = y_ref[...]
    yn = yb / jnp.maximum(
        jnp.sqrt(jnp.sum(yb * yb, axis=0, keepdims=True)), 1e-12)
    a2 = jnp.sum(xn * xn, axis=1, keepdims=True)
    b2 = jnp.sum(yn * yn, axis=0, keepdims=True)
    dot = jax.lax.dot_general(
        xn, yn, (((1,), (0,)), ((), ())),
        preferred_element_type=jnp.float32)
    d2 = a2 + b2 - 2.0 * dot
    s = jnp.sqrt(jnp.maximum(d2, 0.0)) + rel_ref[...]

    col = jax.lax.broadcasted_iota(jnp.int32, s.shape, 1)
    lane = jax.lax.broadcasted_iota(jnp.int32, (s.shape[0], _OUT_W), 1)
    idxs = jnp.zeros((s.shape[0], _OUT_W), jnp.int32)
    big = jnp.int32(2 ** 30)
    for i in range(_K):
        m = jnp.min(s, axis=1, keepdims=True)
        t = jnp.where(s == m, col, big)
        idx = jnp.min(t, axis=1, keepdims=True)
        s = jnp.where(t == idx, jnp.inf, s)
        idxs = jnp.where(lane == i, idx, idxs)
    out_ref[...] = idxs


def _build_call(n, c, interpret=False):
    grid = (n // _BLOCK_R,)
    return pl.pallas_call(
        _knn_block,
        grid=grid,
        in_specs=[
            pl.BlockSpec((_BLOCK_R, c), lambda i: (i, 0)),
            pl.BlockSpec((c, n), lambda i: (0, 0)),
            pl.BlockSpec((_BLOCK_R, n), lambda i: (i, 0)),
        ],
        out_specs=pl.BlockSpec((_BLOCK_R, _OUT_W), lambda i: (i, 0)),
        out_shape=jax.ShapeDtypeStruct((n, _OUT_W), jnp.int32),
        interpret=interpret,
    )


def kernel(x, y, relative_pos):
    b, c, n, _ = x.shape
    xt = jnp.transpose(x.reshape(c, n))
    ys = y.reshape(c, n)
    rel = relative_pos.reshape(n, n)
    out = _build_call(n, c)(xt, ys, rel)
    nn_idx = out[:, :_K].reshape(b, n, _K)
    center_idx = jnp.broadcast_to(
        jnp.arange(n, dtype=jnp.int32)[None, :, None], (b, n, _K))
    return jnp.stack((nn_idx, center_idx), axis=0)
